# baseline (speedup 1.0000x reference)
import jax
import jax.numpy as jnp
from jax.experimental import pallas as pl
from jax.experimental.pallas import tpu as pltpu

N, M = 65536, 512
EPS = 1e-16

_NB = 32
_R = N // _NB
_NRES = 10
_DN = (((1,), (1,)), ((), ()))


def _fused_kernel(beta_ref, g_ref, gamma_ref, s_ref, mem_ref, wprev_ref,
                  k_ref, e_ref, a_ref, out_ref, r_ref, w_s, wg_s, acc_s,
                  res_s):
    j = pl.program_id(0)

    @pl.when(j < _NB)
    def _stats():
        memE = mem_ref[...] + EPS
        kk = k_ref[...] + EPS
        dot = jax.lax.dot_general(
            kk, memE, _DN, preferred_element_type=jnp.float32)
        ones = jnp.ones((1, M), jnp.float32)
        ssq = jax.lax.dot_general(
            ones, memE * memE, _DN, preferred_element_type=jnp.float32)
        knorm = jnp.sqrt(jnp.sum(kk * kk, axis=1, keepdims=True))
        ez = jnp.exp(beta_ref[0] * (dot / (jnp.sqrt(ssq) * knorm + EPS)))
        off = pl.multiple_of(j * _R, _R)
        w_s[:, pl.ds(off, _R)] = ez
        wg_s[:, pl.ds(off, _R)] = wprev_ref[...]
        part = jnp.sum(ez, axis=1, keepdims=True)
        acc_s[:, :1] = jnp.where(j == 0, part, acc_s[:, :1] + part)

    @pl.when((j >= _NB - 1 - _NRES) & (j < _NB - 1))
    def _stash():
        roff = pl.multiple_of((j - (_NB - 1 - _NRES)) * _R, _R)
        res_s[pl.ds(roff, _R), :] = mem_ref[...]

    @pl.when(j == _NB)
    def _address():
        g = g_ref[0]
        gamma = gamma_ref[0]
        s0 = s_ref[0]
        s1 = s_ref[1]
        s2 = s_ref[2]
        gs = g / acc_s[0:1, 0:1]
        wg_s[...] = gs * w_s[...] + (1.0 - g) * wg_s[...]
        wg = wg_s[...]
        m1 = pltpu.roll(wg, 1, axis=1)
        p1 = pltpu.roll(wg, N - 1, axis=1)
        w_s[...] = jnp.exp2(
            gamma * jnp.log2(m1 * s0 + wg * s1 + p1 * s2))
        w_s[...] = w_s[...] * (1.0 / (jnp.sum(w_s[...], axis=1, keepdims=True) + EPS))
        r_ref[...] = jnp.zeros_like(r_ref)

    def _rw_body(m, jj):
        off = pl.multiple_of(jj * _R, _R)
        w = w_s[:, pl.ds(off, _R)].T
        wm = m * w
        r_ref[...] += jnp.sum(wm, axis=0, keepdims=True)
        out_ref[...] = m - wm * e_ref[...] + w * a_ref[...]

    @pl.when((j >= _NB) & (j < 2 * _NB - _NRES))
    def _readwrite():
        jj = jnp.where(j == _NB, _NB - 1, j - _NB - 1)
        _rw_body(mem_ref[...], jj)

    @pl.when(j >= 2 * _NB - _NRES)
    def _readwrite_res():
        roff = pl.multiple_of((j - (2 * _NB - _NRES)) * _R, _R)
        _rw_body(res_s[pl.ds(roff, _R), :], j - _NB - 1)


def kernel(mem, k, beta, g, s, gamma, w_prev, e, a):
    k2 = k.reshape(1, M)
    e2 = e.reshape(1, M)
    a2 = a.reshape(1, M)
    new_mem, r_row = pl.pallas_call(
        _fused_kernel,
        grid=(2 * _NB,),
        in_specs=[
            pl.BlockSpec(memory_space=pltpu.SMEM),
            pl.BlockSpec(memory_space=pltpu.SMEM),
            pl.BlockSpec(memory_space=pltpu.SMEM),
            pl.BlockSpec(memory_space=pltpu.SMEM),
            pl.BlockSpec((_R, M), lambda j: (
                jnp.where(j <= _NB, jnp.minimum(j, _NB - 1),
                          jnp.minimum(j - _NB - 1, _NB - 2 - _NRES)), 0)),
            pl.BlockSpec((1, _R), lambda j: (0, jnp.minimum(j, _NB - 1))),
            pl.BlockSpec((1, M), lambda j: (0, 0)),
            pl.BlockSpec((1, M), lambda j: (0, 0)),
            pl.BlockSpec((1, M), lambda j: (0, 0)),
        ],
        out_specs=[
            pl.BlockSpec((_R, M), lambda j: (
                jnp.where(j <= _NB, _NB - 1, j - _NB - 1), 0)),
            pl.BlockSpec((1, M), lambda j: (0, 0)),
        ],
        out_shape=[
            jax.ShapeDtypeStruct((N, M), jnp.float32),
            jax.ShapeDtypeStruct((1, M), jnp.float32),
        ],
        scratch_shapes=[
            pltpu.VMEM((1, N), jnp.float32),
            pltpu.VMEM((1, N), jnp.float32),
            pltpu.VMEM((1, 128), jnp.float32),
            pltpu.VMEM((_NRES * _R, M), jnp.float32),
        ],
        compiler_params=pltpu.CompilerParams(
            dimension_semantics=("arbitrary",),
            vmem_limit_bytes=63 * 1024 * 1024,
        ),
        name="ntm_fused",
    )(beta.reshape(1), g.reshape(1), gamma.reshape(1), s,
      mem, w_prev.reshape(1, N), k2, e2, a2)

    return r_row.reshape(M), new_mem


# speedup vs baseline: 1.0581x; 1.0581x over previous
import jax
import jax.numpy as jnp
from jax.experimental import pallas as pl
from jax.experimental.pallas import tpu as pltpu

N, M = 65536, 512
EPS = 1e-16

_NB = 16
_R = N // _NB
_NRES = 2
_DN = (((1,), (1,)), ((), ()))


def _fused_kernel(beta_ref, g_ref, gamma_ref, s_ref, mem_ref, wprev_ref,
                  k_ref, e_ref, a_ref, out_ref, r_ref, w_s, wg_s, acc_s,
                  res_s):
    j = pl.program_id(0)

    @pl.when(j < _NB)
    def _stats():
        memE = mem_ref[...] + EPS
        kk = k_ref[...] + EPS
        dot = jax.lax.dot_general(
            kk, memE, _DN, preferred_element_type=jnp.float32)
        ones = jnp.ones((1, M), jnp.float32)
        ssq = jax.lax.dot_general(
            ones, memE * memE, _DN, preferred_element_type=jnp.float32)
        knorm = jnp.sqrt(jnp.sum(kk * kk, axis=1, keepdims=True))
        ez = jnp.exp(beta_ref[0] * (dot / (jnp.sqrt(ssq) * knorm + EPS)))
        off = pl.multiple_of(j * _R, _R)
        w_s[:, pl.ds(off, _R)] = ez
        wg_s[:, pl.ds(off, _R)] = wprev_ref[...]
        part = jnp.sum(ez, axis=1, keepdims=True)
        acc_s[:, :1] = jnp.where(j == 0, part, acc_s[:, :1] + part)

    @pl.when((j >= _NB - 1 - _NRES) & (j < _NB - 1))
    def _stash():
        roff = pl.multiple_of((j - (_NB - 1 - _NRES)) * _R, _R)
        res_s[pl.ds(roff, _R), :] = mem_ref[...]

    @pl.when(j == _NB)
    def _address():
        g = g_ref[0]
        gamma = gamma_ref[0]
        s0 = s_ref[0]
        s1 = s_ref[1]
        s2 = s_ref[2]
        gs = g / acc_s[0:1, 0:1]
        wg_s[...] = gs * w_s[...] + (1.0 - g) * wg_s[...]
        wg = wg_s[...]
        m1 = pltpu.roll(wg, 1, axis=1)
        p1 = pltpu.roll(wg, N - 1, axis=1)
        w_s[...] = jnp.exp2(
            gamma * jnp.log2(m1 * s0 + wg * s1 + p1 * s2))
        w_s[...] = w_s[...] * (1.0 / (jnp.sum(w_s[...], axis=1, keepdims=True) + EPS))
        r_ref[...] = jnp.zeros_like(r_ref)

    def _rw_body(m, jj):
        off = pl.multiple_of(jj * _R, _R)
        w = w_s[:, pl.ds(off, _R)].T
        wm = m * w
        r_ref[...] += jnp.sum(wm, axis=0, keepdims=True)
        out_ref[...] = m - wm * e_ref[...] + w * a_ref[...]

    @pl.when((j >= _NB) & (j < 2 * _NB - _NRES))
    def _readwrite():
        jj = jnp.where(j == _NB, _NB - 1, j - _NB - 1)
        _rw_body(mem_ref[...], jj)

    @pl.when(j >= 2 * _NB - _NRES)
    def _readwrite_res():
        roff = pl.multiple_of((j - (2 * _NB - _NRES)) * _R, _R)
        _rw_body(res_s[pl.ds(roff, _R), :], j - _NB - 1)


def kernel(mem, k, beta, g, s, gamma, w_prev, e, a):
    k2 = k.reshape(1, M)
    e2 = e.reshape(1, M)
    a2 = a.reshape(1, M)
    new_mem, r_row = pl.pallas_call(
        _fused_kernel,
        grid=(2 * _NB,),
        in_specs=[
            pl.BlockSpec(memory_space=pltpu.SMEM),
            pl.BlockSpec(memory_space=pltpu.SMEM),
            pl.BlockSpec(memory_space=pltpu.SMEM),
            pl.BlockSpec(memory_space=pltpu.SMEM),
            pl.BlockSpec((_R, M), lambda j: (
                jnp.where(j <= _NB, jnp.minimum(j, _NB - 1),
                          jnp.minimum(j - _NB - 1, _NB - 2 - _NRES)), 0)),
            pl.BlockSpec((1, _R), lambda j: (0, jnp.minimum(j, _NB - 1))),
            pl.BlockSpec((1, M), lambda j: (0, 0)),
            pl.BlockSpec((1, M), lambda j: (0, 0)),
            pl.BlockSpec((1, M), lambda j: (0, 0)),
        ],
        out_specs=[
            pl.BlockSpec((_R, M), lambda j: (
                jnp.where(j <= _NB, _NB - 1, j - _NB - 1), 0)),
            pl.BlockSpec((1, M), lambda j: (0, 0)),
        ],
        out_shape=[
            jax.ShapeDtypeStruct((N, M), jnp.float32),
            jax.ShapeDtypeStruct((1, M), jnp.float32),
        ],
        scratch_shapes=[
            pltpu.VMEM((1, N), jnp.float32),
            pltpu.VMEM((1, N), jnp.float32),
            pltpu.VMEM((1, 128), jnp.float32),
            pltpu.VMEM((_NRES * _R, M), jnp.float32),
        ],
        compiler_params=pltpu.CompilerParams(
            dimension_semantics=("arbitrary",),
            vmem_limit_bytes=63 * 1024 * 1024,
        ),
        name="ntm_fused",
    )(beta.reshape(1), g.reshape(1), gamma.reshape(1), s,
      mem, w_prev.reshape(1, N), k2, e2, a2)

    return r_row.reshape(M), new_mem
